# baseline (speedup 1.0000x reference)
import jax
import jax.numpy as jnp
from jax.experimental import pallas as pl
from jax.experimental.pallas import tpu as pltpu

_H = 128
_NA = 3


def _dot(x, w):
    return jax.lax.dot_general(
        x, w, (((1,), (0,)), ((), ())), preferred_element_type=jnp.float32
    )


def _fused_kernel(s_ref, a_ref, W1_ref, b1_ref, Wg_ref, bg_ref, W2_ref, b2_ref,
                  Wha_ref, bha_ref, Whb_ref, bhb_ref, q_ref):
    x = jnp.concatenate(
        [s_ref[0], s_ref[1], s_ref[2], a_ref[0], a_ref[1], a_ref[2]], axis=1
    )
    fc1 = jnp.maximum(_dot(x, W1_ref[...]) + b1_ref[...], 0.0)

    xw = _dot(fc1, Wg_ref[...])
    m = (xw[0:1, :] + xw[1:2, :] + xw[2:3, :]) * (1.0 / 3.0)
    rows = jax.lax.broadcasted_iota(jnp.int32, (xw.shape[0], 1), 0)
    is_first = pl.program_id(0) == 0
    xw = jnp.where(jnp.logical_and(is_first, rows < _NA), m, xw)

    g = jnp.maximum(xw + bg_ref[...], 0.0) + fc1
    x2 = jnp.maximum(_dot(g, W2_ref[...]) + b2_ref[...], 0.0)

    h = jnp.maximum(_dot(x2, Wha_ref[...]) + bha_ref[...], 0.0)
    q_ref[...] = _dot(h, Whb_ref[...]) + bhb_ref[...]


def kernel(s, a, W1, b1, Wg, bg, W2, b2, Wq1a, bq1a, Wq1b, bq1b, Wq2a, bq2a,
           Wq2b, bq2b):
    n = s.shape[1]
    obs = s.shape[2]
    act = a.shape[2]

    block = 1000 if n % 1000 == 0 else n
    grid = n // block

    Wha = jnp.concatenate([Wq1a, Wq2a], axis=1)
    bha = jnp.concatenate([bq1a, bq2a], axis=0).reshape(1, 2 * _H)
    Whb = jnp.concatenate(
        [
            jnp.concatenate([Wq1b, jnp.zeros_like(Wq1b)], axis=1),
            jnp.concatenate([jnp.zeros_like(Wq2b), Wq2b], axis=1),
        ],
        axis=0,
    )
    bhb = jnp.concatenate([bq1b, bq2b], axis=0).reshape(1, 2)

    b1r = b1.reshape(1, _H)
    bgr = bg.reshape(1, _H)
    b2r = b2.reshape(1, _H)

    wspec = pl.BlockSpec(lambda i: (0, 0))

    q = pl.pallas_call(
        _fused_kernel,
        grid=(grid,),
        in_specs=[
            pl.BlockSpec((_NA, block, obs), lambda i: (0, i, 0)),
            pl.BlockSpec((_NA, block, act), lambda i: (0, i, 0)),
            pl.BlockSpec(W1.shape, lambda i: (0, 0)),
            pl.BlockSpec((1, _H), lambda i: (0, 0)),
            pl.BlockSpec(Wg.shape, lambda i: (0, 0)),
            pl.BlockSpec((1, _H), lambda i: (0, 0)),
            pl.BlockSpec(W2.shape, lambda i: (0, 0)),
            pl.BlockSpec((1, _H), lambda i: (0, 0)),
            pl.BlockSpec((_H, 2 * _H), lambda i: (0, 0)),
            pl.BlockSpec((1, 2 * _H), lambda i: (0, 0)),
            pl.BlockSpec((2 * _H, 2), lambda i: (0, 0)),
            pl.BlockSpec((1, 2), lambda i: (0, 0)),
        ],
        out_specs=pl.BlockSpec((block, 2), lambda i: (i, 0)),
        out_shape=jax.ShapeDtypeStruct((n, 2), jnp.float32),
        compiler_params=pltpu.CompilerParams(
            dimension_semantics=("parallel",),
        ),
    )(s, a, W1, b1r, Wg, bgr, W2, b2r, Wha, bha, Whb, bhb)

    return (q[:, 0:1], q[:, 1:2])


# speedup vs baseline: 1.1349x; 1.1349x over previous
import jax
import jax.numpy as jnp
from jax.experimental import pallas as pl
from jax.experimental.pallas import tpu as pltpu

_H = 128
_NA = 3


def _dot(x, w):
    return jax.lax.dot_general(
        x, w, (((1,), (0,)), ((), ())), preferred_element_type=jnp.float32
    )


def _fused_kernel(s_ref, a_ref, W1_ref, b1_ref, Wg_ref, bg_ref, W2_ref, b2_ref,
                  Wha_ref, bha_ref, Whb_ref, bhb_ref, q_ref):
    x = jnp.concatenate(
        [s_ref[0], s_ref[1], s_ref[2], a_ref[0], a_ref[1], a_ref[2]], axis=1
    )
    fc1 = jnp.maximum(_dot(x, W1_ref[...]) + b1_ref[...], 0.0)

    xw = _dot(fc1, Wg_ref[...])
    m = (xw[0:1, :] + xw[1:2, :] + xw[2:3, :]) * (1.0 / 3.0)
    rows = jax.lax.broadcasted_iota(jnp.int32, (xw.shape[0], 1), 0)
    is_first = pl.program_id(0) == 0
    xw = jnp.where(jnp.logical_and(is_first, rows < _NA), m, xw)

    g = jnp.maximum(xw + bg_ref[...], 0.0) + fc1
    x2 = jnp.maximum(_dot(g, W2_ref[...]) + b2_ref[...], 0.0)

    h = jnp.maximum(_dot(x2, Wha_ref[...]) + bha_ref[...], 0.0)
    q_ref[...] = _dot(h, Whb_ref[...]) + bhb_ref[...]


def kernel(s, a, W1, b1, Wg, bg, W2, b2, Wq1a, bq1a, Wq1b, bq1b, Wq2a, bq2a,
           Wq2b, bq2b):
    n = s.shape[1]
    obs = s.shape[2]
    act = a.shape[2]

    block = 4000 if n % 4000 == 0 else n
    grid = n // block

    Wha = jnp.concatenate([Wq1a, Wq2a], axis=1)
    bha = jnp.concatenate([bq1a, bq2a], axis=0).reshape(1, 2 * _H)
    Whb = jnp.concatenate(
        [
            jnp.concatenate([Wq1b, jnp.zeros_like(Wq1b)], axis=1),
            jnp.concatenate([jnp.zeros_like(Wq2b), Wq2b], axis=1),
        ],
        axis=0,
    )
    bhb = jnp.concatenate([bq1b, bq2b], axis=0).reshape(1, 2)

    b1r = b1.reshape(1, _H)
    bgr = bg.reshape(1, _H)
    b2r = b2.reshape(1, _H)

    wspec = pl.BlockSpec(lambda i: (0, 0))

    q = pl.pallas_call(
        _fused_kernel,
        grid=(grid,),
        in_specs=[
            pl.BlockSpec((_NA, block, obs), lambda i: (0, i, 0)),
            pl.BlockSpec((_NA, block, act), lambda i: (0, i, 0)),
            pl.BlockSpec(W1.shape, lambda i: (0, 0)),
            pl.BlockSpec((1, _H), lambda i: (0, 0)),
            pl.BlockSpec(Wg.shape, lambda i: (0, 0)),
            pl.BlockSpec((1, _H), lambda i: (0, 0)),
            pl.BlockSpec(W2.shape, lambda i: (0, 0)),
            pl.BlockSpec((1, _H), lambda i: (0, 0)),
            pl.BlockSpec((_H, 2 * _H), lambda i: (0, 0)),
            pl.BlockSpec((1, 2 * _H), lambda i: (0, 0)),
            pl.BlockSpec((2 * _H, 2), lambda i: (0, 0)),
            pl.BlockSpec((1, 2), lambda i: (0, 0)),
        ],
        out_specs=pl.BlockSpec((block, 2), lambda i: (i, 0)),
        out_shape=jax.ShapeDtypeStruct((n, 2), jnp.float32),
        compiler_params=pltpu.CompilerParams(
            dimension_semantics=("parallel",),
        ),
    )(s, a, W1, b1r, Wg, bgr, W2, b2r, Wha, bha, Whb, bhb)

    return (q[:, 0:1], q[:, 1:2])


# speedup vs baseline: 1.2847x; 1.1320x over previous
import jax
import jax.numpy as jnp
from jax.experimental import pallas as pl
from jax.experimental.pallas import tpu as pltpu

_H = 128
_NA = 3


def _dot(x, w):
    return jax.lax.dot_general(
        x, w, (((1,), (0,)), ((), ())), preferred_element_type=jnp.float32
    )


def _fused_kernel(s_ref, a_ref, W1_ref, b1_ref, Wg_ref, bg_ref, W2_ref, b2_ref,
                  Wha_ref, bha_ref, Whb_ref, bhb_ref, q1_ref, q2_ref):
    x = jnp.concatenate(
        [s_ref[0], s_ref[1], s_ref[2], a_ref[0], a_ref[1], a_ref[2]], axis=1
    )
    fc1 = jnp.maximum(_dot(x, W1_ref[...]) + b1_ref[...], 0.0)

    xw = _dot(fc1, Wg_ref[...])
    m = (xw[0:1, :] + xw[1:2, :] + xw[2:3, :]) * (1.0 / 3.0)
    rows = jax.lax.broadcasted_iota(jnp.int32, (xw.shape[0], 1), 0)
    is_first = pl.program_id(0) == 0
    xw = jnp.where(jnp.logical_and(is_first, rows < _NA), m, xw)

    g = jnp.maximum(xw + bg_ref[...], 0.0) + fc1
    x2 = jnp.maximum(_dot(g, W2_ref[...]) + b2_ref[...], 0.0)

    h = jnp.maximum(_dot(x2, Wha_ref[...]) + bha_ref[...], 0.0)
    q = _dot(h, Whb_ref[...]) + bhb_ref[...]
    q1_ref[...] = q[:, 0:1]
    q2_ref[...] = q[:, 1:2]


def kernel(s, a, W1, b1, Wg, bg, W2, b2, Wq1a, bq1a, Wq1b, bq1b, Wq2a, bq2a,
           Wq2b, bq2b):
    n = s.shape[1]
    obs = s.shape[2]
    act = a.shape[2]

    block = 4000 if n % 4000 == 0 else n
    grid = n // block

    Wha = jnp.concatenate([Wq1a, Wq2a], axis=1)
    bha = jnp.concatenate([bq1a, bq2a], axis=0).reshape(1, 2 * _H)
    Whb = jnp.concatenate(
        [
            jnp.concatenate([Wq1b, jnp.zeros_like(Wq1b)], axis=1),
            jnp.concatenate([jnp.zeros_like(Wq2b), Wq2b], axis=1),
        ],
        axis=0,
    )
    bhb = jnp.concatenate([bq1b, bq2b], axis=0).reshape(1, 2)

    b1r = b1.reshape(1, _H)
    bgr = bg.reshape(1, _H)
    b2r = b2.reshape(1, _H)

    wspec = pl.BlockSpec(lambda i: (0, 0))

    q = pl.pallas_call(
        _fused_kernel,
        grid=(grid,),
        in_specs=[
            pl.BlockSpec((_NA, block, obs), lambda i: (0, i, 0)),
            pl.BlockSpec((_NA, block, act), lambda i: (0, i, 0)),
            pl.BlockSpec(W1.shape, lambda i: (0, 0)),
            pl.BlockSpec((1, _H), lambda i: (0, 0)),
            pl.BlockSpec(Wg.shape, lambda i: (0, 0)),
            pl.BlockSpec((1, _H), lambda i: (0, 0)),
            pl.BlockSpec(W2.shape, lambda i: (0, 0)),
            pl.BlockSpec((1, _H), lambda i: (0, 0)),
            pl.BlockSpec((_H, 2 * _H), lambda i: (0, 0)),
            pl.BlockSpec((1, 2 * _H), lambda i: (0, 0)),
            pl.BlockSpec((2 * _H, 2), lambda i: (0, 0)),
            pl.BlockSpec((1, 2), lambda i: (0, 0)),
        ],
        out_specs=[
            pl.BlockSpec((block, 1), lambda i: (i, 0)),
            pl.BlockSpec((block, 1), lambda i: (i, 0)),
        ],
        out_shape=[
            jax.ShapeDtypeStruct((n, 1), jnp.float32),
            jax.ShapeDtypeStruct((n, 1), jnp.float32),
        ],
        compiler_params=pltpu.CompilerParams(
            dimension_semantics=("parallel",),
        ),
    )(s, a, W1, b1r, Wg, bgr, W2, b2r, Wha, bha, Whb, bhb)

    return (q[0], q[1])
